# baseline (speedup 1.0000x reference)
import jax
import jax.numpy as jnp
from jax.experimental import pallas as pl
from jax.experimental.pallas import tpu as pltpu

_LP = 256
_LQ = 200
_BB = 64


def _rosa_block(u_ref, x_ref, v_ref, o_ref):
    x = x_ref[...]
    v = v_ref[...]
    u = u_ref[0, 0]

    tq_i = jax.lax.broadcasted_iota(jnp.int32, (1, _LQ, _LP), 1)
    tk_i = jax.lax.broadcasted_iota(jnp.int32, (1, _LQ, _LP), 2)
    tk = tk_i.astype(jnp.float32)
    tkm = jnp.where(tk_i < tq_i, tk, -1.0)

    xq = x[:, :_LQ]
    eq = xq[:, :, None] == x[:, None, :]
    last = jnp.max(jnp.where(eq, tkm, -1.0), axis=2)

    onehot = tk == last[:, :, None]
    gathered = jnp.sum(jnp.where(onehot, v[:, None, :], 0.0), axis=2)
    o_ref[:, :_LQ] = jnp.where(last >= 0.0, gathered, u)


def kernel(x, v, u):
    B, L = x.shape
    x32 = x.astype(jnp.int32)
    xp = jnp.full((B, _LP), -1, dtype=jnp.int32).at[:, :L].set(x32)
    vp = jnp.zeros((B, _LP), dtype=jnp.float32).at[:, :L].set(v)
    u_arr = jnp.full((1, 1), u, dtype=jnp.float32)

    out = pl.pallas_call(
        _rosa_block,
        grid=(B // _BB,),
        in_specs=[
            pl.BlockSpec(memory_space=pltpu.SMEM),
            pl.BlockSpec((_BB, _LP), lambda i: (i, 0)),
            pl.BlockSpec((_BB, _LP), lambda i: (i, 0)),
        ],
        out_specs=pl.BlockSpec((_BB, _LP), lambda i: (i, 0)),
        out_shape=jax.ShapeDtypeStruct((B, _LP), jnp.float32),
    )(u_arr, xp, vp)
    return out[:, :L]


# speedup vs baseline: 7.0600x; 7.0600x over previous
import jax
import jax.numpy as jnp
from jax.experimental import pallas as pl
from jax.experimental.pallas import tpu as pltpu

_LQ = 200
_BBL = 256


def _rosa_block(u_ref, x_ref, v_ref, o_ref):
    xq = x_ref[...]
    vq = v_ref[...]
    u = u_ref[0, 0]

    rows = jax.lax.broadcasted_iota(jnp.int32, (_LQ, 1), 0)
    out = jnp.full((_LQ, _BBL), u, dtype=jnp.float32)
    o_ref[...] = out
    for tp in range(_LQ - 1):
        lo = ((tp + 1) // 8) * 8
        xc = xq[tp:tp + 1, :]
        vc = vq[tp:tp + 1, :]
        m = (xq[lo:, :] == xc) & (rows[lo:, :] > tp)
        o_ref[lo:, :] = jnp.where(m, vc, o_ref[lo:, :])


def kernel(x, v, u):
    B, L = x.shape
    xT = x.astype(jnp.int32).T
    vT = v.T
    u_arr = jnp.full((1, 1), u, dtype=jnp.float32)

    out = pl.pallas_call(
        _rosa_block,
        grid=(B // _BBL,),
        in_specs=[
            pl.BlockSpec(memory_space=pltpu.SMEM),
            pl.BlockSpec((L, _BBL), lambda i: (0, i)),
            pl.BlockSpec((L, _BBL), lambda i: (0, i)),
        ],
        out_specs=pl.BlockSpec((L, _BBL), lambda i: (0, i)),
        out_shape=jax.ShapeDtypeStruct((L, B), jnp.float32),
    )(u_arr, xT, vT)
    return out.T
